# baseline (speedup 1.0000x reference)
import functools

import jax
import jax.numpy as jnp
from jax.experimental import pallas as pl
from jax.experimental.pallas import tpu as pltpu

_TEMPERATURE = 0.07
_BIG = 1e9
_DELTA = 1.5


def _body(emb_ref, embT_ref, lab_row_ref, lab_col_ref, out_ref, acc_ref,
          *, block_r, n_blocks, b_total):
    i = pl.program_id(0)

    @pl.when(i == 0)
    def _init():
        acc_ref[0] = 0.0
        acc_ref[1] = 0.0

    sim = jnp.dot(emb_ref[...], embT_ref[...],
                  preferred_element_type=jnp.float32)

    lr = lab_row_ref[...]
    lc = lab_col_ref[...]
    eq = lr == lc

    col = jax.lax.broadcasted_iota(jnp.int32, (block_r, b_total), 1)
    row = jax.lax.broadcasted_iota(jnp.int32, (block_r, b_total), 0)
    eye = col == row + i * block_r

    pos = eq & (~eye)
    posf = pos.astype(jnp.float32)
    pos_cnt = jnp.sum(posf, axis=1, keepdims=True)
    pos_sum = jnp.sum(posf * sim, axis=1, keepdims=True)
    pos_min = jnp.min(jnp.where(pos, sim, _BIG), axis=1, keepdims=True)

    neg = ~eq
    semi = neg & (sim < pos_min)
    has_semi = jnp.sum(semi.astype(jnp.float32), axis=1, keepdims=True) > 0.0
    thr = jnp.where(has_semi, pos_min, _BIG)
    v = jnp.where(neg & (sim < thr), sim, -_BIG)

    n_fold = 8
    wslot = b_total // n_fold
    f = [v[:, k * wslot:(k + 1) * wslot] for k in range(n_fold)]
    net = [(0, 1), (2, 3), (4, 5), (6, 7),
           (0, 2), (1, 3), (4, 6), (5, 7),
           (1, 2), (5, 6),
           (0, 4), (1, 5), (2, 6), (3, 7),
           (2, 4), (3, 5),
           (1, 2), (3, 4), (5, 6)]
    for a, b in net:
        hi_ab = jnp.maximum(f[a], f[b])
        lo_ab = jnp.minimum(f[a], f[b])
        f[a], f[b] = hi_ab, lo_ab

    m1 = jnp.max(f[0], axis=1, keepdims=True)
    rem = jnp.full((block_r, 1), 16.0, dtype=jnp.float32)
    sum_exp = jnp.zeros((block_r, 1), dtype=jnp.float32)
    for _ in range(16):
        m = jnp.max(f[0], axis=1, keepdims=True)
        e = f[0] == m
        c = jnp.sum(e.astype(jnp.float32), axis=1, keepdims=True)
        take = jnp.minimum(c, rem)
        sum_exp = sum_exp + take * jnp.exp((m - m1) / _TEMPERATURE)
        rem = rem - take
        for k in range(n_fold - 1):
            f[k] = jnp.where(e, f[k + 1], f[k])
        f[n_fold - 1] = jnp.where(e, -_BIG, f[n_fold - 1])

    neg_lse = m1 / _TEMPERATURE + jnp.log(sum_exp)

    pos_mean = pos_sum / jnp.maximum(pos_cnt, 1.0)
    loss_i = -pos_mean / _TEMPERATURE + neg_lse
    neg_cnt = (b_total - 1.0) - pos_cnt
    valid = (pos_cnt > 0.0) & (neg_cnt > 0.0)

    acc_ref[0] += jnp.sum(jnp.where(valid, loss_i, 0.0))
    acc_ref[1] += jnp.sum(valid.astype(jnp.float32))

    @pl.when(i == n_blocks - 1)
    def _fin():
        out_ref[0] = acc_ref[0] / jnp.maximum(acc_ref[1], 1.0)


@jax.jit
def kernel(embeddings, labels):
    b_total, d = embeddings.shape
    block_r = 256
    n_blocks = b_total // block_r

    embT = embeddings.T
    lab_row = labels.reshape(b_total, 1)
    lab_col = labels.reshape(1, b_total)

    body = functools.partial(_body, block_r=block_r, n_blocks=n_blocks,
                             b_total=b_total)
    out = pl.pallas_call(
        body,
        grid=(n_blocks,),
        in_specs=[
            pl.BlockSpec((block_r, d), lambda i: (i, 0)),
            pl.BlockSpec((d, b_total), lambda i: (0, 0)),
            pl.BlockSpec((block_r, 1), lambda i: (i, 0)),
            pl.BlockSpec((1, b_total), lambda i: (0, 0)),
        ],
        out_specs=pl.BlockSpec(memory_space=pltpu.SMEM),
        out_shape=jax.ShapeDtypeStruct((1,), jnp.float32),
        scratch_shapes=[pltpu.SMEM((2,), jnp.float32)],
    )(embeddings, embT, lab_row, lab_col)
    return out[0]


# speedup vs baseline: 1.5024x; 1.5024x over previous
import functools

import jax
import jax.numpy as jnp
from jax.experimental import pallas as pl
from jax.experimental.pallas import tpu as pltpu

_TEMPERATURE = 0.07
_BIG = 1e9
_N_HARD = 16


def _batcher_pairs(n, keep):
    pairs = []
    p = 1
    while p < n:
        k = p
        while k >= 1:
            for j in range(k % p, n - k, 2 * k):
                for i in range(0, min(k, n - j - k)):
                    if (i + j) // (2 * p) == (i + j + k) // (2 * p):
                        pairs.append((i + j, i + j + k))
            k //= 2
        p *= 2
    needed = set(range(keep))
    kept = []
    for a, b in reversed(pairs):
        if a in needed or b in needed:
            kept.append((a, b))
            needed.add(a)
            needed.add(b)
    return list(reversed(kept))


def _body(emb_ref, embT_ref, lab_row_ref, lab_col_ref, out_ref, acc_ref,
          *, block_r, n_blocks, b_total):
    i = pl.program_id(0)

    @pl.when(i == 0)
    def _init():
        acc_ref[0] = 0.0
        acc_ref[1] = 0.0

    sim = jnp.dot(emb_ref[...], embT_ref[...],
                  preferred_element_type=jnp.float32)

    lr = lab_row_ref[...]
    lc = lab_col_ref[...]
    eq = lr == lc

    col = jax.lax.broadcasted_iota(jnp.int32, (block_r, b_total), 1)
    row = jax.lax.broadcasted_iota(jnp.int32, (block_r, b_total), 0)
    eye = col == row + i * block_r

    pos = eq & (~eye)
    posf = eq.astype(jnp.float32) - eye.astype(jnp.float32)
    pos_cnt = jnp.sum(posf, axis=1, keepdims=True)
    pos_sum = jnp.sum(posf * sim, axis=1, keepdims=True)
    pos_min = jnp.min(jnp.where(pos, sim, _BIG), axis=1, keepdims=True)

    neg = ~eq
    neg_min = jnp.min(jnp.where(neg, sim, _BIG), axis=1, keepdims=True)
    has_semi = neg_min < pos_min
    thr = jnp.where(has_semi, pos_min, _BIG)
    v = jnp.where(neg & (sim < thr), sim, -_BIG)

    n_fold = 32
    wslot = b_total // n_fold
    f = [v[:, k * wslot:(k + 1) * wslot] for k in range(n_fold)]
    for a, b in _batcher_pairs(n_fold, _N_HARD):
        hi_ab = jnp.maximum(f[a], f[b])
        lo_ab = jnp.minimum(f[a], f[b])
        f[a], f[b] = hi_ab, lo_ab

    m1 = jnp.max(f[0], axis=1, keepdims=True)
    ms = []
    cs = []
    for t in range(_N_HARD):
        m = m1 if t == 0 else jnp.max(f[0], axis=1, keepdims=True)
        e = f[0] == m
        c = jnp.sum(e.astype(jnp.float32), axis=1, keepdims=True)
        ms.append(m)
        cs.append(c)
        depth = min(n_fold - 1, 15 - t)
        for k in range(depth):
            f[k] = jnp.where(e, f[k + 1], f[k])
        if depth == n_fold - 1:
            f[n_fold - 1] = jnp.where(e, -_BIG, f[n_fold - 1])

    mm = jnp.concatenate(ms, axis=1)
    cc = jnp.concatenate(cs, axis=1)
    s = cc
    for sh in (1, 2, 4, 8):
        s = s + jnp.pad(s, ((0, 0), (sh, 0)))[:, :_N_HARD]
    prev = s - cc
    take = jnp.minimum(cc, jnp.maximum(float(_N_HARD) - prev, 0.0))
    w = jnp.exp((mm - m1) / _TEMPERATURE)
    sum_exp = jnp.sum(take * w, axis=1, keepdims=True)

    neg_lse = m1 / _TEMPERATURE + jnp.log(sum_exp)

    pos_mean = pos_sum / jnp.maximum(pos_cnt, 1.0)
    loss_i = -pos_mean / _TEMPERATURE + neg_lse
    neg_cnt = (b_total - 1.0) - pos_cnt
    valid = (pos_cnt > 0.0) & (neg_cnt > 0.0)

    acc_ref[0] += jnp.sum(jnp.where(valid, loss_i, 0.0))
    acc_ref[1] += jnp.sum(valid.astype(jnp.float32))

    @pl.when(i == n_blocks - 1)
    def _fin():
        out_ref[0] = acc_ref[0] / jnp.maximum(acc_ref[1], 1.0)


@jax.jit
def kernel(embeddings, labels):
    b_total, d = embeddings.shape
    block_r = 256
    n_blocks = b_total // block_r

    embT = embeddings.T
    lab_row = labels.reshape(b_total, 1)
    lab_col = labels.reshape(1, b_total)

    body = functools.partial(_body, block_r=block_r, n_blocks=n_blocks,
                             b_total=b_total)
    out = pl.pallas_call(
        body,
        grid=(n_blocks,),
        in_specs=[
            pl.BlockSpec((block_r, d), lambda i: (i, 0)),
            pl.BlockSpec((d, b_total), lambda i: (0, 0)),
            pl.BlockSpec((block_r, 1), lambda i: (i, 0)),
            pl.BlockSpec((1, b_total), lambda i: (0, 0)),
        ],
        out_specs=pl.BlockSpec(memory_space=pltpu.SMEM),
        out_shape=jax.ShapeDtypeStruct((1,), jnp.float32),
        scratch_shapes=[pltpu.SMEM((2,), jnp.float32)],
    )(embeddings, embT, lab_row, lab_col)
    return out[0]


# speedup vs baseline: 1.6566x; 1.1027x over previous
import functools

import jax
import jax.numpy as jnp
from jax.experimental import pallas as pl
from jax.experimental.pallas import tpu as pltpu

_TEMPERATURE = 0.07
_BIG = 1e9
_N_HARD = 16


def _batcher_pairs(n, keep):
    pairs = []
    p = 1
    while p < n:
        k = p
        while k >= 1:
            for j in range(k % p, n - k, 2 * k):
                for i in range(0, min(k, n - j - k)):
                    if (i + j) // (2 * p) == (i + j + k) // (2 * p):
                        pairs.append((i + j, i + j + k))
            k //= 2
        p *= 2
    needed = set(range(keep))
    kept = []
    for a, b in reversed(pairs):
        if a in needed or b in needed:
            kept.append((a, b))
            needed.add(a)
            needed.add(b)
    return list(reversed(kept))


def _body(emb_ref, embT_ref, lab_row_ref, lab_col_ref, out_ref, acc_ref,
          *, block_r, n_blocks, b_total):
    i = pl.program_id(0)

    @pl.when(i == 0)
    def _init():
        acc_ref[0] = 0.0
        acc_ref[1] = 0.0

    sim = jnp.dot(emb_ref[...], embT_ref[...],
                  preferred_element_type=jnp.float32)

    lr = lab_row_ref[...]
    lc = lab_col_ref[...]
    eq = lr == lc

    col = jax.lax.broadcasted_iota(jnp.int32, (block_r, b_total), 1)
    row = jax.lax.broadcasted_iota(jnp.int32, (block_r, b_total), 0)
    eye = col == row + i * block_r

    simd = jnp.where(eye, _BIG, sim)
    eqf = eq.astype(jnp.float32)
    self_dot = jnp.sum(emb_ref[...] * emb_ref[...], axis=1, keepdims=True)
    pos_cnt = jnp.sum(eqf, axis=1, keepdims=True) - 1.0
    pos_sum = jnp.sum(eqf * sim, axis=1, keepdims=True) - self_dot
    pos_min = jnp.min(jnp.where(eq, simd, _BIG), axis=1, keepdims=True)
    neg_min = jnp.min(jnp.where(eq, _BIG, sim), axis=1, keepdims=True)
    has_semi = neg_min < pos_min
    thr = jnp.where(has_semi, pos_min, _BIG)
    v = jnp.where(eq | (sim >= thr), -_BIG, sim)

    n_fold = 32
    wslot = b_total // n_fold
    f = [v[:, k * wslot:(k + 1) * wslot] for k in range(n_fold)]
    for a, b in _batcher_pairs(n_fold, _N_HARD):
        hi_ab = jnp.maximum(f[a], f[b])
        lo_ab = jnp.minimum(f[a], f[b])
        f[a], f[b] = hi_ab, lo_ab

    m1 = jnp.max(f[0], axis=1, keepdims=True)
    ms = []
    cs = []
    for t in range(_N_HARD):
        m = m1 if t == 0 else jnp.max(f[0], axis=1, keepdims=True)
        e = f[0] == m
        c = jnp.sum(e.astype(jnp.float32), axis=1, keepdims=True)
        ms.append(m)
        cs.append(c)
        depth = min(n_fold - 1, 15 - t)
        for k in range(depth):
            f[k] = jnp.where(e, f[k + 1], f[k])
        if depth == n_fold - 1:
            f[n_fold - 1] = jnp.where(e, -_BIG, f[n_fold - 1])

    mm = jnp.concatenate(ms, axis=1)
    cc = jnp.concatenate(cs, axis=1)
    s = cc
    for sh in (1, 2, 4, 8):
        s = s + jnp.pad(s, ((0, 0), (sh, 0)))[:, :_N_HARD]
    prev = s - cc
    take = jnp.minimum(cc, jnp.maximum(float(_N_HARD) - prev, 0.0))
    w = jnp.exp((mm - m1) / _TEMPERATURE)
    sum_exp = jnp.sum(take * w, axis=1, keepdims=True)

    neg_lse = m1 / _TEMPERATURE + jnp.log(sum_exp)

    pos_mean = pos_sum / jnp.maximum(pos_cnt, 1.0)
    loss_i = -pos_mean / _TEMPERATURE + neg_lse
    neg_cnt = (b_total - 1.0) - pos_cnt
    valid = (pos_cnt > 0.0) & (neg_cnt > 0.0)

    acc_ref[0] += jnp.sum(jnp.where(valid, loss_i, 0.0))
    acc_ref[1] += jnp.sum(valid.astype(jnp.float32))

    @pl.when(i == n_blocks - 1)
    def _fin():
        out_ref[0] = acc_ref[0] / jnp.maximum(acc_ref[1], 1.0)


@jax.jit
def kernel(embeddings, labels):
    b_total, d = embeddings.shape
    block_r = 256
    n_blocks = b_total // block_r

    embT = embeddings.T
    lab_row = labels.reshape(b_total, 1)
    lab_col = labels.reshape(1, b_total)

    body = functools.partial(_body, block_r=block_r, n_blocks=n_blocks,
                             b_total=b_total)
    out = pl.pallas_call(
        body,
        grid=(n_blocks,),
        in_specs=[
            pl.BlockSpec((block_r, d), lambda i: (i, 0)),
            pl.BlockSpec((d, b_total), lambda i: (0, 0)),
            pl.BlockSpec((block_r, 1), lambda i: (i, 0)),
            pl.BlockSpec((1, b_total), lambda i: (0, 0)),
        ],
        out_specs=pl.BlockSpec(memory_space=pltpu.SMEM),
        out_shape=jax.ShapeDtypeStruct((1,), jnp.float32),
        scratch_shapes=[pltpu.SMEM((2,), jnp.float32)],
    )(embeddings, embT, lab_row, lab_col)
    return out[0]
